# Initial kernel scaffold; baseline (speedup 1.0000x reference)
import jax
import jax.numpy as jnp
from jax.experimental import pallas as pl


def kernel(sequence, group_id, emb, conv_w, proj_w, score_w, score_b):
    raise NotImplementedError("write your pallas kernel here")


# speedup vs baseline: 23.2681x; 23.2681x over previous
import functools

import jax
import jax.numpy as jnp
from jax.experimental import pallas as pl
from jax.experimental.pallas import tpu as pltpu

B, S, D, K, V, GMAX = 16, 2048, 128, 4, 256, 512
CH = 512
NCH = S // CH
SP = S + 8


def _gbst_kernel(gid_ref, idxT_ref, emb_ref, wk_ref, projT_ref,
                 swT_ref, sb_ref, out_ref, esp_ref):
    f32 = jnp.float32

    for c in range(NCH):
        seq_col = idxT_ref[0, pl.ds(c * CH, CH), 0:1]
        vio = jax.lax.broadcasted_iota(jnp.int32, (CH, V), 1)
        ohe = (seq_col == vio).astype(f32)
        esp_ref[pl.ds(c * CH, CH), :] = jnp.dot(
            ohe, emb_ref[...], preferred_element_type=f32)
    esp_ref[pl.ds(S, SP - S), :] = jnp.zeros((SP - S, D), f32)

    conv = esp_ref[pl.ds(0, S), :] * wk_ref[0:1, :]
    for k in range(1, K):
        conv = conv + esp_ref[pl.ds(k, S), :] * wk_ref[k:k + 1, :]
    es2 = jnp.dot(conv, projT_ref[...], preferred_element_type=f32)

    reps = [es2]
    for l in range(K - 1):
        sums = jnp.zeros((GMAX, D), f32)
        cnt_row = jnp.zeros((1, GMAX), f32)
        for c in range(NCH):
            gid_row = gid_ref[0, l:l + 1, pl.ds(c * CH, CH)]
            gid_col = idxT_ref[0, pl.ds(c * CH, CH), l + 1:l + 2]
            gio = jax.lax.broadcasted_iota(jnp.int32, (GMAX, CH), 0)
            oh = (gid_row == gio + 1).astype(f32)
            gio2 = jax.lax.broadcasted_iota(jnp.int32, (CH, GMAX), 1)
            ohT = (gid_col == gio2 + 1).astype(f32)
            es2_c = es2[c * CH:(c + 1) * CH, :]
            sums = sums + jnp.dot(oh, es2_c, preferred_element_type=f32)
            cnt_row = cnt_row + jnp.sum(ohT, axis=0, keepdims=True)
        gi = jax.lax.broadcasted_iota(jnp.int32, (GMAX, GMAX), 0)
        gj = jax.lax.broadcasted_iota(jnp.int32, (GMAX, GMAX), 1)
        tri = (gi <= gj).astype(f32)
        cum_row = jnp.dot(cnt_row, tri, preferred_element_type=f32)
        recip_row = 1.0 / jnp.maximum(cnt_row, 1.0)
        lo_row = cum_row - cnt_row
        rep_chunks = []
        for c in range(NCH):
            sio = jax.lax.broadcasted_iota(jnp.int32, (CH, GMAX), 0)
            s_pos = (sio + c * CH).astype(f32)
            oh2 = jnp.where((s_pos >= lo_row) & (s_pos < cum_row),
                            recip_row, 0.0).astype(f32)
            rep_chunks.append(jnp.dot(oh2, sums, preferred_element_type=f32))
        reps.append(jnp.concatenate(rep_chunks, axis=0))

    neg = -jnp.finfo(f32).max
    bias = sb_ref[0, 0]
    scores = []
    for k in range(K):
        sc = jnp.dot(reps[k], swT_ref[...], preferred_element_type=f32) + bias
        mask = idxT_ref[0, :, k:k + 1] == 0
        scores.append(jnp.where(mask, neg, sc))
    m = jnp.maximum(jnp.maximum(scores[0], scores[1]),
                    jnp.maximum(scores[2], scores[3]))
    exps = [jnp.exp(sc - m) for sc in scores]
    denom = exps[0] + exps[1] + exps[2] + exps[3]
    out = (reps[0] * exps[0] + reps[1] * exps[1]
           + reps[2] * exps[2] + reps[3] * exps[3]) / denom
    out_ref[0] = out


@jax.jit
def kernel(sequence, group_id, emb, conv_w, proj_w, score_w, score_b):
    f32 = jnp.float32
    idxT = jnp.concatenate(
        [sequence[:, :, None], jnp.transpose(group_id, (0, 2, 1))], axis=2)
    wk = conv_w[:, 0, :].T
    projT = proj_w[:, :, 0].T
    swT = score_w.T
    sb = score_b.reshape(1, 1).astype(f32)

    grid = (B,)
    return pl.pallas_call(
        _gbst_kernel,
        grid=grid,
        in_specs=[
            pl.BlockSpec((1, K - 1, S), lambda b: (b, 0, 0)),
            pl.BlockSpec((1, S, K), lambda b: (b, 0, 0)),
            pl.BlockSpec((V, D), lambda b: (0, 0)),
            pl.BlockSpec((K, D), lambda b: (0, 0)),
            pl.BlockSpec((D, D), lambda b: (0, 0)),
            pl.BlockSpec((D, 1), lambda b: (0, 0)),
            pl.BlockSpec((1, 1), lambda b: (0, 0)),
        ],
        out_specs=pl.BlockSpec((1, S, D), lambda b: (b, 0, 0)),
        out_shape=jax.ShapeDtypeStruct((B, S, D), f32),
        scratch_shapes=[pltpu.VMEM((SP, D), f32)],
        compiler_params=pltpu.CompilerParams(
            dimension_semantics=("arbitrary",)),
    )(group_id, idxT, emb, wk, projT, swT, sb)


# speedup vs baseline: 26.9068x; 1.1564x over previous
import functools

import jax
import jax.numpy as jnp
from jax.experimental import pallas as pl
from jax.experimental.pallas import tpu as pltpu

B, S, D, K, V, GMAX = 16, 2048, 128, 4, 256, 512
SP = S + 8


def _gbst_kernel(idxT_ref, emb_ref, wk_ref, projT_ref,
                 swT_ref, sb_ref, out_ref, esp_ref):
    f32 = jnp.float32

    seq_col = idxT_ref[0, :, 0:1]
    vio = jax.lax.broadcasted_iota(jnp.int32, (S, V), 1)
    ohe = (seq_col == vio).astype(f32)
    esp_ref[pl.ds(0, S), :] = jnp.dot(
        ohe, emb_ref[...], preferred_element_type=f32)
    esp_ref[pl.ds(S, SP - S), :] = jnp.zeros((SP - S, D), f32)

    conv = esp_ref[pl.ds(0, S), :] * wk_ref[0:1, :]
    for k in range(1, K):
        conv = conv + esp_ref[pl.ds(k, S), :] * wk_ref[k:k + 1, :]
    es2 = jnp.dot(conv, projT_ref[...], preferred_element_type=f32)

    reps = [es2]
    for l in range(K - 1):
        gid_col = idxT_ref[0, :, l + 1:l + 2]
        gio = jax.lax.broadcasted_iota(jnp.int32, (S, GMAX), 1)
        ohT = (gid_col == gio + 1).astype(f32)
        sums = jax.lax.dot_general(
            ohT, es2, (((0,), (0,)), ((), ())),
            preferred_element_type=f32)
        cnt_row = jnp.sum(ohT, axis=0, keepdims=True)
        gi = jax.lax.broadcasted_iota(jnp.int32, (GMAX, GMAX), 0)
        gj = jax.lax.broadcasted_iota(jnp.int32, (GMAX, GMAX), 1)
        tri = (gi <= gj).astype(f32)
        cum_row = jnp.dot(cnt_row, tri, preferred_element_type=f32)
        recip_row = 1.0 / jnp.maximum(cnt_row, 1.0)
        lo_row = cum_row - cnt_row
        sio = jax.lax.broadcasted_iota(jnp.int32, (S, GMAX), 0)
        s_pos = sio.astype(f32)
        oh2 = jnp.where((s_pos >= lo_row) & (s_pos < cum_row),
                        recip_row, 0.0).astype(f32)
        reps.append(jnp.dot(oh2, sums, preferred_element_type=f32))

    neg = -jnp.finfo(f32).max
    bias = sb_ref[0, 0]
    scores = []
    for k in range(K):
        sc = jnp.dot(reps[k], swT_ref[...], preferred_element_type=f32) + bias
        mask = idxT_ref[0, :, k:k + 1] == 0
        scores.append(jnp.where(mask, neg, sc))
    m = jnp.maximum(jnp.maximum(scores[0], scores[1]),
                    jnp.maximum(scores[2], scores[3]))
    exps = [jnp.exp(sc - m) for sc in scores]
    denom = exps[0] + exps[1] + exps[2] + exps[3]
    out = (reps[0] * exps[0] + reps[1] * exps[1]
           + reps[2] * exps[2] + reps[3] * exps[3]) / denom
    out_ref[0] = out


@jax.jit
def kernel(sequence, group_id, emb, conv_w, proj_w, score_w, score_b):
    f32 = jnp.float32
    idxT = jnp.concatenate(
        [sequence[:, :, None], jnp.transpose(group_id, (0, 2, 1))], axis=2)
    wk = conv_w[:, 0, :].T
    projT = proj_w[:, :, 0].T
    swT = score_w.T
    sb = score_b.reshape(1, 1).astype(f32)

    grid = (B,)
    return pl.pallas_call(
        _gbst_kernel,
        grid=grid,
        in_specs=[
            pl.BlockSpec((1, S, K), lambda b: (b, 0, 0)),
            pl.BlockSpec((V, D), lambda b: (0, 0)),
            pl.BlockSpec((K, D), lambda b: (0, 0)),
            pl.BlockSpec((D, D), lambda b: (0, 0)),
            pl.BlockSpec((D, 1), lambda b: (0, 0)),
            pl.BlockSpec((1, 1), lambda b: (0, 0)),
        ],
        out_specs=pl.BlockSpec((1, S, D), lambda b: (b, 0, 0)),
        out_shape=jax.ShapeDtypeStruct((B, S, D), f32),
        scratch_shapes=[pltpu.VMEM((SP, D), f32)],
        compiler_params=pltpu.CompilerParams(
            dimension_semantics=("parallel",)),
    )(idxT, emb, wk, projT, swT, sb)
